# baseline (speedup 1.0000x reference)
import functools

import jax
import jax.numpy as jnp
from jax import lax
from jax.experimental import pallas as pl
from jax.experimental.pallas import tpu as pltpu
from jax.experimental.pallas import tpu_sc as plsc

B = 16384
E = 64
N_NEG = 20
NC = 2
NS = 16
NW = NC * NS
BPW = B // NW
CB = 32
NCHUNK = BPW // CB
NIDX = CB * N_NEG


def _sc_body(tgt_hbm, ctx_hbm, noise_hbm, in_hbm, out_hbm,
             ctxo_hbm, noiseo_hbm,
             tgt_idx, ctx_idx, noise_idx, t_buf, c_buf, n_buf,
             ctx_dots, noise_dots, sem):
    wid = lax.axis_index("s") * NC + lax.axis_index("c")
    iota = lax.iota(jnp.int32, 16)

    pltpu.sync_copy(tgt_hbm.at[pl.ds(wid * NCHUNK, NCHUNK)], tgt_idx)
    pltpu.sync_copy(ctx_hbm.at[pl.ds(wid * NCHUNK, NCHUNK)], ctx_idx)
    nrows = NIDX * NCHUNK // 128
    pltpu.sync_copy(noise_hbm.at[pl.ds(wid * nrows, nrows)], noise_idx)

    def chunk_body(c, carry):
        pltpu.async_copy(in_hbm.at[tgt_idx.at[c]], t_buf, sem).wait()
        pltpu.async_copy(out_hbm.at[ctx_idx.at[c]], c_buf, sem).wait()
        for k in range(NIDX // 128):
            pltpu.async_copy(out_hbm.at[noise_idx.at[c * (NIDX // 128) + k]],
                             n_buf.at[pl.ds(k * 128, 128)], sem).wait()

        for g in range(CB // 16):
            rows = g * 16 + iota
            rows20 = rows * N_NEG

            def dbody(i, accs):
                acc_c = accs[0]
                acc_n = list(accs[1:])
                for u in range(4):
                    d = 4 * i + u
                    col = jnp.full((16,), 0, jnp.int32) + d
                    tT = plsc.load_gather(t_buf, [rows, col])
                    cT = plsc.load_gather(c_buf, [rows, col])
                    acc_c = acc_c + tT * cT
                    for s in range(N_NEG):
                        nT = plsc.load_gather(n_buf, [rows20 + s, col])
                        acc_n[s] = acc_n[s] + nT * tT
                return (acc_c, *acc_n)

            zero = jnp.zeros((16,), jnp.float32)
            accs = lax.fori_loop(0, E // 4, dbody,
                                 tuple(zero for _ in range(1 + N_NEG)))
            base = c * CB + g * 16
            ctx_dots[pl.ds(base, 16)] = accs[0]
            for s in range(N_NEG):
                noise_dots[s, pl.ds(base, 16)] = accs[1 + s]
        return carry

    lax.fori_loop(0, NCHUNK, chunk_body, 0)

    pltpu.sync_copy(ctx_dots, ctxo_hbm.at[wid])
    pltpu.sync_copy(noise_dots, noiseo_hbm.at[wid])


_sc_kernel = functools.partial(
    pl.kernel,
    out_type=(
        jax.ShapeDtypeStruct((NW, BPW), jnp.float32),
        jax.ShapeDtypeStruct((NW, N_NEG, BPW), jnp.float32),
    ),
    mesh=plsc.VectorSubcoreMesh(core_axis_name="c", subcore_axis_name="s",
                                num_cores=NC, num_subcores=NS),
    compiler_params=pltpu.CompilerParams(needs_layout_passes=False,
                                         use_tc_tiling_on_sc=False),
    scratch_types=[
        pltpu.VMEM((NCHUNK, CB), jnp.int32),
        pltpu.VMEM((NCHUNK, CB), jnp.int32),
        pltpu.VMEM((NIDX * NCHUNK // 128, 128), jnp.int32),
        pltpu.VMEM((CB, E), jnp.float32),
        pltpu.VMEM((CB, E), jnp.float32),
        pltpu.VMEM((NIDX, E), jnp.float32),
        pltpu.VMEM((BPW,), jnp.float32),
        pltpu.VMEM((N_NEG, BPW), jnp.float32),
        pltpu.SemaphoreType.DMA,
    ],
)(_sc_body)


def _tc_body(ctx_ref, noise_ref, out_ref):
    cd = ctx_ref[...]
    nd = noise_ref[...]
    total = (jnp.sum(jnp.log(jax.nn.sigmoid(cd)))
             + jnp.sum(jnp.log(jax.nn.sigmoid(-nd))))
    out_ref[...] = jnp.full((1, 1), -total * (1.0 / B), jnp.float32)


_tc_kernel = pl.pallas_call(
    _tc_body,
    out_shape=jax.ShapeDtypeStruct((1, 1), jnp.float32),
)


@jax.jit
def kernel(target, context, noise_words, in_embed, out_embed):
    tgt2 = target.astype(jnp.int32).reshape(NW * NCHUNK, CB)
    ctx2 = context.astype(jnp.int32).reshape(NW * NCHUNK, CB)
    noise2 = noise_words.astype(jnp.int32).reshape(B * N_NEG // 128, 128)
    ctx_dots, noise_dots = _sc_kernel(tgt2, ctx2, noise2, in_embed, out_embed)
    loss = _tc_kernel(ctx_dots.reshape(B // 128, 128),
                      noise_dots.reshape(B * N_NEG // 128, 128))
    return loss[0, 0]


# speedup vs baseline: 1.0586x; 1.0586x over previous
import functools

import jax
import jax.numpy as jnp
from jax import lax
from jax.experimental import pallas as pl
from jax.experimental.pallas import tpu as pltpu
from jax.experimental.pallas import tpu_sc as plsc

B = 16384
E = 64
N_NEG = 20
NC = 2
NS = 16
NW = NC * NS
BPW = B // NW
CB = 32
NCHUNK = BPW // CB
NIDX = CB * N_NEG


def _sc_body(tgt_hbm, ctx_hbm, noise_hbm, in_hbm, out_hbm,
             ctxo_hbm, noiseo_hbm,
             tgt_idx, ctx_idx, noise_idx, t_buf, c_buf, n_buf,
             ctx_dots, noise_dots, sem0, sem1):
    wid = lax.axis_index("s") * NC + lax.axis_index("c")
    iota = lax.iota(jnp.int32, 16)

    pltpu.sync_copy(tgt_hbm.at[pl.ds(wid * NCHUNK, NCHUNK)], tgt_idx)
    pltpu.sync_copy(ctx_hbm.at[pl.ds(wid * NCHUNK, NCHUNK)], ctx_idx)
    nrows = NIDX * NCHUNK // 128
    pltpu.sync_copy(noise_hbm.at[pl.ds(wid * nrows, nrows)], noise_idx)

    def fire(c, b, sem):
        pltpu.async_copy(in_hbm.at[tgt_idx.at[c]], t_buf.at[b], sem)
        pltpu.async_copy(out_hbm.at[ctx_idx.at[c]], c_buf.at[b], sem)
        for k in range(NIDX // 128):
            pltpu.async_copy(out_hbm.at[noise_idx.at[c * (NIDX // 128) + k]],
                             n_buf.at[b, pl.ds(k * 128, 128)], sem)

    def drain(b, sem):
        pltpu.make_async_copy(in_hbm.at[pl.ds(0, CB)], t_buf.at[b], sem).wait()
        pltpu.make_async_copy(out_hbm.at[pl.ds(0, CB)], c_buf.at[b], sem).wait()
        for k in range(NIDX // 128):
            pltpu.make_async_copy(out_hbm.at[pl.ds(0, 128)],
                                  n_buf.at[b, pl.ds(k * 128, 128)], sem).wait()

    def compute(c, b):
        t_slot = t_buf.at[b]
        c_slot = c_buf.at[b]
        n_slot = n_buf.at[b]
        for g in range(CB // 16):
            rows = g * 16 + iota
            rows20 = rows * N_NEG

            def dbody(i, accs):
                acc_c = accs[0]
                acc_n = list(accs[1:])
                for u in range(4):
                    d = 4 * i + u
                    col = jnp.full((16,), 0, jnp.int32) + d
                    tT = plsc.load_gather(t_slot, [rows, col])
                    cT = plsc.load_gather(c_slot, [rows, col])
                    acc_c = acc_c + tT * cT
                    for s in range(N_NEG):
                        nT = plsc.load_gather(n_slot, [rows20 + s, col])
                        acc_n[s] = acc_n[s] + nT * tT
                return (acc_c, *acc_n)

            zero = jnp.zeros((16,), jnp.float32)
            accs = lax.fori_loop(0, E // 4, dbody,
                                 tuple(zero for _ in range(1 + N_NEG)))
            base = c * CB + g * 16
            ctx_dots[pl.ds(base, 16)] = accs[0]
            for s in range(N_NEG):
                noise_dots[s, pl.ds(base, 16)] = accs[1 + s]

    fire(0, 0, sem0)

    def pipe_body(j, carry):
        c = 2 * j
        fire(c + 1, 1, sem1)
        drain(0, sem0)
        compute(c, 0)

        @pl.when(j < NCHUNK // 2 - 1)
        def _():
            fire(c + 2, 0, sem0)

        drain(1, sem1)
        compute(c + 1, 1)
        return carry

    lax.fori_loop(0, NCHUNK // 2, pipe_body, 0)

    pltpu.sync_copy(ctx_dots, ctxo_hbm.at[wid])
    pltpu.sync_copy(noise_dots, noiseo_hbm.at[wid])


_sc_kernel = functools.partial(
    pl.kernel,
    out_type=(
        jax.ShapeDtypeStruct((NW, BPW), jnp.float32),
        jax.ShapeDtypeStruct((NW, N_NEG, BPW), jnp.float32),
    ),
    mesh=plsc.VectorSubcoreMesh(core_axis_name="c", subcore_axis_name="s",
                                num_cores=NC, num_subcores=NS),
    compiler_params=pltpu.CompilerParams(needs_layout_passes=False,
                                         use_tc_tiling_on_sc=False),
    scratch_types=[
        pltpu.VMEM((NCHUNK, CB), jnp.int32),
        pltpu.VMEM((NCHUNK, CB), jnp.int32),
        pltpu.VMEM((NIDX * NCHUNK // 128, 128), jnp.int32),
        pltpu.VMEM((2, CB, E), jnp.float32),
        pltpu.VMEM((2, CB, E), jnp.float32),
        pltpu.VMEM((2, NIDX, E), jnp.float32),
        pltpu.VMEM((BPW,), jnp.float32),
        pltpu.VMEM((N_NEG, BPW), jnp.float32),
        pltpu.SemaphoreType.DMA,
        pltpu.SemaphoreType.DMA,
    ],
)(_sc_body)


def _tc_body(ctx_ref, noise_ref, out_ref):
    cd = ctx_ref[...]
    nd = noise_ref[...]
    total = (jnp.sum(jnp.log(jax.nn.sigmoid(cd)))
             + jnp.sum(jnp.log(jax.nn.sigmoid(-nd))))
    out_ref[...] = jnp.full((1, 1), -total * (1.0 / B), jnp.float32)


_tc_kernel = pl.pallas_call(
    _tc_body,
    out_shape=jax.ShapeDtypeStruct((1, 1), jnp.float32),
)


@jax.jit
def kernel(target, context, noise_words, in_embed, out_embed):
    tgt2 = target.astype(jnp.int32).reshape(NW * NCHUNK, CB)
    ctx2 = context.astype(jnp.int32).reshape(NW * NCHUNK, CB)
    noise2 = noise_words.astype(jnp.int32).reshape(B * N_NEG // 128, 128)
    ctx_dots, noise_dots = _sc_kernel(tgt2, ctx2, noise2, in_embed, out_embed)
    loss = _tc_kernel(ctx_dots.reshape(B // 128, 128),
                      noise_dots.reshape(B * N_NEG // 128, 128))
    return loss[0, 0]


# speedup vs baseline: 1.2617x; 1.1918x over previous
import functools

import jax
import jax.numpy as jnp
from jax import lax
from jax.experimental import pallas as pl
from jax.experimental.pallas import tpu as pltpu
from jax.experimental.pallas import tpu_sc as plsc

B = 16384
E = 64
N_NEG = 20
NC = 2
NS = 16
NW = NC * NS
BPW = B // NW
CB = 32
NCHUNK = BPW // CB
NIDX = CB * N_NEG


def _sc_body(tgt_hbm, ctx_hbm, noise_hbm, in_hbm, out_hbm,
             ctxo_hbm, noiseo_hbm,
             tgt_idx, ctx_idx, noise_idx, t_buf, c_buf, n_buf,
             ctx_dots, noise_dots, sem0, sem1):
    wid = lax.axis_index("s") * NC + lax.axis_index("c")
    iota = lax.iota(jnp.int32, 16)

    pltpu.sync_copy(tgt_hbm.at[pl.ds(wid * NCHUNK, NCHUNK)], tgt_idx)
    pltpu.sync_copy(ctx_hbm.at[pl.ds(wid * NCHUNK, NCHUNK)], ctx_idx)
    nrows = NIDX * NCHUNK // 128
    pltpu.sync_copy(noise_hbm.at[pl.ds(wid * nrows, nrows)], noise_idx)

    def fire(c, b, sem):
        pltpu.async_copy(in_hbm.at[tgt_idx.at[c]], t_buf.at[b], sem)
        pltpu.async_copy(out_hbm.at[ctx_idx.at[c]], c_buf.at[b], sem)
        for k in range(NIDX // 128):
            pltpu.async_copy(out_hbm.at[noise_idx.at[c * (NIDX // 128) + k]],
                             n_buf.at[b, pl.ds(k * 128, 128)], sem)

    def drain(b, sem):
        pltpu.make_async_copy(in_hbm.at[pl.ds(0, CB)], t_buf.at[b], sem).wait()
        pltpu.make_async_copy(out_hbm.at[pl.ds(0, CB)], c_buf.at[b], sem).wait()
        for k in range(NIDX // 128):
            pltpu.make_async_copy(out_hbm.at[pl.ds(0, 128)],
                                  n_buf.at[b, pl.ds(k * 128, 128)], sem).wait()

    def compute(c, b):
        t_slot = t_buf.at[b]
        c_slot = c_buf.at[b]
        n_slot = n_buf.at[b]

        lane15 = iota == 15

        def jbody(j, carry):
            t = [t_slot[j, pl.ds(16 * k, 16)] for k in range(E // 16)]

            def dot_store(row_ref, r, dots_ref, pos):
                x = [row_ref[r, pl.ds(16 * k, 16)] for k in range(E // 16)]
                p = (t[0] * x[0] + t[1] * x[1]) + (t[2] * x[2] + t[3] * x[3])
                cum = plsc.cumsum(p)
                idx = jnp.full((16,), 0, jnp.int32) + pos
                plsc.store_scatter(dots_ref, [idx], cum, mask=lane15)

            dot_store(c_slot, j, ctx_dots, c * CB + j)
            for s in range(N_NEG):
                dot_store(n_slot, j * N_NEG + s, noise_dots,
                          s * BPW + c * CB + j)
            return carry

        lax.fori_loop(0, CB, jbody, 0)

    fire(0, 0, sem0)

    def pipe_body(j, carry):
        c = 2 * j
        fire(c + 1, 1, sem1)
        drain(0, sem0)
        compute(c, 0)

        @pl.when(j < NCHUNK // 2 - 1)
        def _():
            fire(c + 2, 0, sem0)

        drain(1, sem1)
        compute(c + 1, 1)
        return carry

    lax.fori_loop(0, NCHUNK // 2, pipe_body, 0)

    pltpu.sync_copy(ctx_dots, ctxo_hbm.at[wid])
    pltpu.sync_copy(noise_dots, noiseo_hbm.at[wid])


_sc_kernel = functools.partial(
    pl.kernel,
    out_type=(
        jax.ShapeDtypeStruct((NW, BPW), jnp.float32),
        jax.ShapeDtypeStruct((NW, N_NEG * BPW), jnp.float32),
    ),
    mesh=plsc.VectorSubcoreMesh(core_axis_name="c", subcore_axis_name="s",
                                num_cores=NC, num_subcores=NS),
    compiler_params=pltpu.CompilerParams(needs_layout_passes=False,
                                         use_tc_tiling_on_sc=False),
    scratch_types=[
        pltpu.VMEM((NCHUNK, CB), jnp.int32),
        pltpu.VMEM((NCHUNK, CB), jnp.int32),
        pltpu.VMEM((NIDX * NCHUNK // 128, 128), jnp.int32),
        pltpu.VMEM((2, CB, E), jnp.float32),
        pltpu.VMEM((2, CB, E), jnp.float32),
        pltpu.VMEM((2, NIDX, E), jnp.float32),
        pltpu.VMEM((BPW,), jnp.float32),
        pltpu.VMEM((N_NEG * BPW,), jnp.float32),
        pltpu.SemaphoreType.DMA,
        pltpu.SemaphoreType.DMA,
    ],
)(_sc_body)


def _tc_body(ctx_ref, noise_ref, out_ref):
    cd = ctx_ref[...]
    nd = noise_ref[...]
    total = (jnp.sum(jnp.log(jax.nn.sigmoid(cd)))
             + jnp.sum(jnp.log(jax.nn.sigmoid(-nd))))
    out_ref[...] = jnp.full((1, 1), -total * (1.0 / B), jnp.float32)


_tc_kernel = pl.pallas_call(
    _tc_body,
    out_shape=jax.ShapeDtypeStruct((1, 1), jnp.float32),
)


@jax.jit
def kernel(target, context, noise_words, in_embed, out_embed):
    tgt2 = target.astype(jnp.int32).reshape(NW * NCHUNK, CB)
    ctx2 = context.astype(jnp.int32).reshape(NW * NCHUNK, CB)
    noise2 = noise_words.astype(jnp.int32).reshape(B * N_NEG // 128, 128)
    ctx_dots, noise_dots = _sc_kernel(tgt2, ctx2, noise2, in_embed, out_embed)
    loss = _tc_kernel(ctx_dots.reshape(B // 128, 128),
                      noise_dots.reshape(B * N_NEG // 128, 128))
    return loss[0, 0]


# speedup vs baseline: 2.1909x; 1.7365x over previous
import functools

import jax
import jax.numpy as jnp
from jax import lax
from jax.experimental import pallas as pl
from jax.experimental.pallas import tpu as pltpu
from jax.experimental.pallas import tpu_sc as plsc

VOCAB = 1000000
B = 16384
E = 64
N_NEG = 20
NC = 2
NS = 16
NW = NC * NS
BPW = B // NW
CB = 16
NCHUNK = BPW // CB
NIDX = CB * N_NEG

VCHUNK = 2048
NBLK = (VOCAB + VCHUNK - 1) // VCHUNK


def _tr_body(a_ref, b_ref, o_ref):
    o_ref[...] = jnp.concatenate([a_ref[...], b_ref[...]], axis=0).T


_tr_kernel = pl.pallas_call(
    _tr_body,
    grid=(NBLK,),
    in_specs=[
        pl.BlockSpec((E, VCHUNK), lambda i: (0, i)),
        pl.BlockSpec((E, VCHUNK), lambda i: (0, i)),
    ],
    out_specs=pl.BlockSpec((VCHUNK, 2 * E), lambda i: (i, 0)),
    out_shape=jax.ShapeDtypeStruct((NBLK * VCHUNK, 2 * E), jnp.float32),
)


def _sc_body(tgt_hbm, ctx_hbm, noise_hbm, tbl_hbm,
             ctxo_hbm, noiseo_hbm,
             tgt_idx, ctx_idx, noise_idx, t_buf, c_buf, n_buf,
             ctx_dots, noise_dots, sem0, sem1):
    wid = lax.axis_index("s") * NC + lax.axis_index("c")
    iota = lax.iota(jnp.int32, 16)

    pltpu.sync_copy(tgt_hbm.at[pl.ds(wid * BPW, BPW)], tgt_idx)
    pltpu.sync_copy(ctx_hbm.at[pl.ds(wid * BPW, BPW)], ctx_idx)
    pltpu.sync_copy(noise_hbm.at[pl.ds(wid * BPW * N_NEG, BPW * N_NEG)],
                    noise_idx)

    def fire(c, b, sem):
        pltpu.async_copy(tbl_hbm.at[tgt_idx.at[pl.ds(c * CB, CB)]],
                         t_buf.at[b], sem)
        pltpu.async_copy(tbl_hbm.at[ctx_idx.at[pl.ds(c * CB, CB)]],
                         c_buf.at[b], sem)
        for off, ln in ((0, 128), (128, 128), (256, 64)):
            pltpu.async_copy(
                tbl_hbm.at[noise_idx.at[pl.ds(c * NIDX + off, ln)]],
                n_buf.at[b, pl.ds(off, ln)], sem)

    def drain(b, sem):
        pltpu.make_async_copy(tbl_hbm.at[pl.ds(0, CB)], t_buf.at[b],
                              sem).wait()
        pltpu.make_async_copy(tbl_hbm.at[pl.ds(0, CB)], c_buf.at[b],
                              sem).wait()
        for off, ln in ((0, 128), (128, 128), (256, 64)):
            pltpu.make_async_copy(tbl_hbm.at[pl.ds(0, ln)],
                                  n_buf.at[b, pl.ds(off, ln)], sem).wait()

    def compute(c, b):
        t_slot = t_buf.at[b]
        c_slot = c_buf.at[b]
        n_slot = n_buf.at[b]
        lane15 = iota == 15

        def jbody(j, carry):
            t = [t_slot[j, pl.ds(16 * k, 16)] for k in range(E // 16)]

            def dot_store(row_ref, r, dots_ref, pos):
                x = [row_ref[r, pl.ds(E + 16 * k, 16)]
                     for k in range(E // 16)]
                p = (t[0] * x[0] + t[1] * x[1]) + (t[2] * x[2] + t[3] * x[3])
                cum = plsc.cumsum(p)
                idx = jnp.full((16,), 0, jnp.int32) + pos
                plsc.store_scatter(dots_ref, [idx], cum, mask=lane15)

            dot_store(c_slot, j, ctx_dots, c * CB + j)
            for s in range(N_NEG):
                dot_store(n_slot, j * N_NEG + s, noise_dots,
                          s * BPW + c * CB + j)
            return carry

        lax.fori_loop(0, CB, jbody, 0)

    fire(0, 0, sem0)

    def pipe_body(j, carry):
        c = 2 * j
        fire(c + 1, 1, sem1)
        drain(0, sem0)
        compute(c, 0)

        @pl.when(j < NCHUNK // 2 - 1)
        def _():
            fire(c + 2, 0, sem0)

        drain(1, sem1)
        compute(c + 1, 1)
        return carry

    lax.fori_loop(0, NCHUNK // 2, pipe_body, 0)

    pltpu.sync_copy(ctx_dots, ctxo_hbm.at[wid])
    pltpu.sync_copy(noise_dots, noiseo_hbm.at[wid])


_sc_kernel = functools.partial(
    pl.kernel,
    out_type=(
        jax.ShapeDtypeStruct((NW, BPW), jnp.float32),
        jax.ShapeDtypeStruct((NW, N_NEG * BPW), jnp.float32),
    ),
    mesh=plsc.VectorSubcoreMesh(core_axis_name="c", subcore_axis_name="s",
                                num_cores=NC, num_subcores=NS),
    compiler_params=pltpu.CompilerParams(needs_layout_passes=False),
    scratch_types=[
        pltpu.VMEM((BPW,), jnp.int32),
        pltpu.VMEM((BPW,), jnp.int32),
        pltpu.VMEM((BPW * N_NEG,), jnp.int32),
        pltpu.VMEM((2, CB, 2 * E), jnp.float32),
        pltpu.VMEM((2, CB, 2 * E), jnp.float32),
        pltpu.VMEM((2, NIDX, 2 * E), jnp.float32),
        pltpu.VMEM((BPW,), jnp.float32),
        pltpu.VMEM((N_NEG * BPW,), jnp.float32),
        pltpu.SemaphoreType.DMA,
        pltpu.SemaphoreType.DMA,
    ],
)(_sc_body)


def _tc_body(ctx_ref, noise_ref, out_ref):
    cd = ctx_ref[...]
    nd = noise_ref[...]
    total = (jnp.sum(jnp.log(jax.nn.sigmoid(cd)))
             + jnp.sum(jnp.log(jax.nn.sigmoid(-nd))))
    out_ref[...] = jnp.full((1, 1), -total * (1.0 / B), jnp.float32)


_tc_kernel = pl.pallas_call(
    _tc_body,
    out_shape=jax.ShapeDtypeStruct((1, 1), jnp.float32),
)


@jax.jit
def kernel(target, context, noise_words, in_embed, out_embed):
    tbl = _tr_kernel(in_embed.T, out_embed.T)
    ctx_dots, noise_dots = _sc_kernel(
        target.astype(jnp.int32),
        context.astype(jnp.int32),
        noise_words.astype(jnp.int32).reshape(B * N_NEG),
        tbl,
    )
    loss = _tc_kernel(ctx_dots.reshape(B // 128, 128),
                      noise_dots.reshape(B * N_NEG // 128, 128))
    return loss[0, 0]


# speedup vs baseline: 2.7243x; 1.2435x over previous
import functools

import jax
import jax.numpy as jnp
from jax import lax
from jax.experimental import pallas as pl
from jax.experimental.pallas import tpu as pltpu
from jax.experimental.pallas import tpu_sc as plsc

VOCAB = 1000000
B = 16384
E = 64
N_NEG = 20
NC = 2
NS = 16
NW = NC * NS
BPW = B // NW
CB = 16
NCHUNK = BPW // CB
NIDX = CB * N_NEG

VCHUNK = 4096
NBLK = (VOCAB + VCHUNK - 1) // VCHUNK


def _tr_body(a_ref, b_ref, o_ref):
    o_ref[...] = jnp.concatenate([a_ref[...], b_ref[...]], axis=0).T


_tr_kernel = pl.pallas_call(
    _tr_body,
    grid=(NBLK,),
    in_specs=[
        pl.BlockSpec((E, VCHUNK), lambda i: (0, i)),
        pl.BlockSpec((E, VCHUNK), lambda i: (0, i)),
    ],
    out_specs=pl.BlockSpec((VCHUNK, 2 * E), lambda i: (i, 0)),
    out_shape=jax.ShapeDtypeStruct((NBLK * VCHUNK, 2 * E), jnp.float32),
)


def _sc_body(tgt_hbm, ctx_hbm, noise_hbm, tbl_hbm,
             ctxo_hbm, noiseo_hbm,
             tgt_idx, ctx_idx, noise_idx, t_buf, c_buf, n_buf,
             ctx_dots, noise_dots, sem0, sem1):
    wid = lax.axis_index("s") * NC + lax.axis_index("c")
    iota = lax.iota(jnp.int32, 16)

    pltpu.sync_copy(tgt_hbm.at[pl.ds(wid * BPW, BPW)], tgt_idx)
    pltpu.sync_copy(ctx_hbm.at[pl.ds(wid * BPW, BPW)], ctx_idx)
    pltpu.sync_copy(noise_hbm.at[pl.ds(wid * BPW * N_NEG, BPW * N_NEG)],
                    noise_idx)

    def fire(c, b, sem):
        pltpu.async_copy(tbl_hbm.at[tgt_idx.at[pl.ds(c * CB, CB)]],
                         t_buf.at[b], sem)
        pltpu.async_copy(tbl_hbm.at[ctx_idx.at[pl.ds(c * CB, CB)]],
                         c_buf.at[b], sem)
        for off, ln in ((0, 128), (128, 128), (256, 64)):
            pltpu.async_copy(
                tbl_hbm.at[noise_idx.at[pl.ds(c * NIDX + off, ln)]],
                n_buf.at[b, pl.ds(off, ln)], sem)

    def drain(b, sem):
        pltpu.make_async_copy(tbl_hbm.at[pl.ds(0, CB)], t_buf.at[b],
                              sem).wait()
        pltpu.make_async_copy(tbl_hbm.at[pl.ds(0, CB)], c_buf.at[b],
                              sem).wait()
        for off, ln in ((0, 128), (128, 128), (256, 64)):
            pltpu.make_async_copy(tbl_hbm.at[pl.ds(0, ln)],
                                  n_buf.at[b, pl.ds(off, ln)], sem).wait()

    def compute(c, b):
        t_slot = t_buf.at[b]
        c_slot = c_buf.at[b]
        n_slot = n_buf.at[b]
        lane15 = iota == 15

        def jbody(j, carry):
            t = [t_slot[j, pl.ds(16 * k, 16)] for k in range(E // 16)]

            def dot_store(row_ref, r, dots_ref, pos):
                x = [row_ref[r, pl.ds(E + 16 * k, 16)]
                     for k in range(E // 16)]
                p = (t[0] * x[0] + t[1] * x[1]) + (t[2] * x[2] + t[3] * x[3])
                cum = plsc.cumsum(p)
                idx = jnp.full((16,), 0, jnp.int32) + pos
                plsc.store_scatter(dots_ref, [idx], cum, mask=lane15)

            dot_store(c_slot, j, ctx_dots, c * CB + j)
            for s in range(N_NEG):
                dot_store(n_slot, j * N_NEG + s, noise_dots,
                          s * BPW + c * CB + j)
            return carry

        lax.fori_loop(0, CB, jbody, 0)

    fire(0, 0, sem0)

    def pipe_body(j, carry):
        c = 2 * j
        fire(c + 1, 1, sem1)
        drain(0, sem0)
        compute(c, 0)

        @pl.when(j < NCHUNK // 2 - 1)
        def _():
            fire(c + 2, 0, sem0)

        drain(1, sem1)
        compute(c + 1, 1)
        return carry

    lax.fori_loop(0, NCHUNK // 2, pipe_body, 0)

    pltpu.sync_copy(ctx_dots, ctxo_hbm.at[wid])
    pltpu.sync_copy(noise_dots, noiseo_hbm.at[wid])


_sc_kernel = functools.partial(
    pl.kernel,
    out_type=(
        jax.ShapeDtypeStruct((NW, BPW), jnp.float32),
        jax.ShapeDtypeStruct((NW, N_NEG * BPW), jnp.float32),
    ),
    mesh=plsc.VectorSubcoreMesh(core_axis_name="c", subcore_axis_name="s",
                                num_cores=NC, num_subcores=NS),
    compiler_params=pltpu.CompilerParams(needs_layout_passes=False),
    scratch_types=[
        pltpu.VMEM((BPW,), jnp.int32),
        pltpu.VMEM((BPW,), jnp.int32),
        pltpu.VMEM((BPW * N_NEG,), jnp.int32),
        pltpu.VMEM((2, CB, 2 * E), jnp.float32),
        pltpu.VMEM((2, CB, 2 * E), jnp.float32),
        pltpu.VMEM((2, NIDX, 2 * E), jnp.float32),
        pltpu.VMEM((BPW,), jnp.float32),
        pltpu.VMEM((N_NEG * BPW,), jnp.float32),
        pltpu.SemaphoreType.DMA,
        pltpu.SemaphoreType.DMA,
    ],
)(_sc_body)


def _tc_body(ctx_ref, noise_ref, out_ref):
    cd = ctx_ref[...]
    nd = noise_ref[...]
    total = (jnp.sum(jnp.log(jax.nn.sigmoid(cd)))
             + jnp.sum(jnp.log(jax.nn.sigmoid(-nd))))
    out_ref[...] = jnp.full((1, 1), -total * (1.0 / B), jnp.float32)


_tc_kernel = pl.pallas_call(
    _tc_body,
    out_shape=jax.ShapeDtypeStruct((1, 1), jnp.float32),
)


@jax.jit
def kernel(target, context, noise_words, in_embed, out_embed):
    tbl = _tr_kernel(in_embed.T, out_embed.T)
    ctx_dots, noise_dots = _sc_kernel(
        target.astype(jnp.int32),
        context.astype(jnp.int32),
        noise_words.astype(jnp.int32).reshape(B * N_NEG),
        tbl,
    )
    loss = _tc_kernel(ctx_dots.reshape(B // 128, 128),
                      noise_dots.reshape(B * N_NEG // 128, 128))
    return loss[0, 0]


# speedup vs baseline: 3.0052x; 1.1031x over previous
import functools

import jax
import jax.numpy as jnp
from jax import lax
from jax.experimental import pallas as pl
from jax.experimental.pallas import tpu as pltpu
from jax.experimental.pallas import tpu_sc as plsc

VOCAB = 1000000
B = 16384
E = 64
N_NEG = 20
NC = 2
NS = 16
NW = NC * NS
BPW = B // NW
CB = 16
NCHUNK = BPW // CB
NIDX = CB * N_NEG

VCHUNK = 8192
NBLK = (VOCAB + VCHUNK - 1) // VCHUNK


def _tr_body(a_ref, b_ref, o_ref):
    o_ref[...] = jnp.concatenate([a_ref[...], b_ref[...]], axis=0).T


_tr_kernel = pl.pallas_call(
    _tr_body,
    grid=(NBLK,),
    in_specs=[
        pl.BlockSpec((E, VCHUNK), lambda i: (0, i)),
        pl.BlockSpec((E, VCHUNK), lambda i: (0, i)),
    ],
    out_specs=pl.BlockSpec((VCHUNK, 2 * E), lambda i: (i, 0)),
    out_shape=jax.ShapeDtypeStruct((NBLK * VCHUNK, 2 * E), jnp.float32),
)


def _sc_body(tgt_hbm, ctx_hbm, noise_hbm, tbl_hbm,
             ctxo_hbm, noiseo_hbm,
             tgt_idx, ctx_idx, noise_idx, t_buf, c_buf, n_buf,
             ctx_dots, noise_dots, sem0, sem1):
    wid = lax.axis_index("s") * NC + lax.axis_index("c")
    iota = lax.iota(jnp.int32, 16)

    pltpu.sync_copy(tgt_hbm.at[pl.ds(wid * BPW, BPW)], tgt_idx)
    pltpu.sync_copy(ctx_hbm.at[pl.ds(wid * BPW, BPW)], ctx_idx)
    pltpu.sync_copy(noise_hbm.at[pl.ds(wid * BPW * N_NEG, BPW * N_NEG)],
                    noise_idx)

    def fire(c, b, sem):
        pltpu.async_copy(tbl_hbm.at[tgt_idx.at[pl.ds(c * CB, CB)]],
                         t_buf.at[b], sem)
        pltpu.async_copy(tbl_hbm.at[ctx_idx.at[pl.ds(c * CB, CB)]],
                         c_buf.at[b], sem)
        for off, ln in ((0, 128), (128, 128), (256, 64)):
            pltpu.async_copy(
                tbl_hbm.at[noise_idx.at[pl.ds(c * NIDX + off, ln)]],
                n_buf.at[b, pl.ds(off, ln)], sem)

    def drain(b, sem):
        pltpu.make_async_copy(tbl_hbm.at[pl.ds(0, CB)], t_buf.at[b],
                              sem).wait()
        pltpu.make_async_copy(tbl_hbm.at[pl.ds(0, CB)], c_buf.at[b],
                              sem).wait()
        for off, ln in ((0, 128), (128, 128), (256, 64)):
            pltpu.make_async_copy(tbl_hbm.at[pl.ds(0, ln)],
                                  n_buf.at[b, pl.ds(off, ln)], sem).wait()

    def compute(c, b):
        t_slot = t_buf.at[b]
        c_slot = c_buf.at[b]
        n_slot = n_buf.at[b]
        lane15 = iota == 15

        def jbody(j, carry):
            t = [t_slot[j, pl.ds(16 * k, 16)] for k in range(E // 16)]

            def dot_store(row_ref, r, dots_ref, pos):
                x = [row_ref[r, pl.ds(E + 16 * k, 16)]
                     for k in range(E // 16)]
                p = (t[0] * x[0] + t[1] * x[1]) + (t[2] * x[2] + t[3] * x[3])
                cum = plsc.cumsum(p)
                idx = jnp.full((16,), 0, jnp.int32) + pos
                plsc.store_scatter(dots_ref, [idx], cum, mask=lane15)

            dot_store(c_slot, j, ctx_dots, c * CB + j)
            for s in range(N_NEG):
                dot_store(n_slot, j * N_NEG + s, noise_dots,
                          s * BPW + c * CB + j)
            return carry

        lax.fori_loop(0, CB, jbody, 0)

    fire(0, 0, sem0)

    def pipe_body(j, carry):
        c = 2 * j
        fire(c + 1, 1, sem1)
        drain(0, sem0)
        compute(c, 0)

        @pl.when(j < NCHUNK // 2 - 1)
        def _():
            fire(c + 2, 0, sem0)

        drain(1, sem1)
        compute(c + 1, 1)
        return carry

    lax.fori_loop(0, NCHUNK // 2, pipe_body, 0)

    pltpu.sync_copy(ctx_dots, ctxo_hbm.at[wid])
    pltpu.sync_copy(noise_dots, noiseo_hbm.at[wid])


_sc_kernel = functools.partial(
    pl.kernel,
    out_type=(
        jax.ShapeDtypeStruct((NW, BPW), jnp.float32),
        jax.ShapeDtypeStruct((NW, N_NEG * BPW), jnp.float32),
    ),
    mesh=plsc.VectorSubcoreMesh(core_axis_name="c", subcore_axis_name="s",
                                num_cores=NC, num_subcores=NS),
    compiler_params=pltpu.CompilerParams(needs_layout_passes=False),
    scratch_types=[
        pltpu.VMEM((BPW,), jnp.int32),
        pltpu.VMEM((BPW,), jnp.int32),
        pltpu.VMEM((BPW * N_NEG,), jnp.int32),
        pltpu.VMEM((2, CB, 2 * E), jnp.float32),
        pltpu.VMEM((2, CB, 2 * E), jnp.float32),
        pltpu.VMEM((2, NIDX, 2 * E), jnp.float32),
        pltpu.VMEM((BPW,), jnp.float32),
        pltpu.VMEM((N_NEG * BPW,), jnp.float32),
        pltpu.SemaphoreType.DMA,
        pltpu.SemaphoreType.DMA,
    ],
)(_sc_body)


def _tc_body(ctx_ref, noise_ref, out_ref):
    cd = ctx_ref[...]
    nd = noise_ref[...]
    total = (jnp.sum(jnp.log(jax.nn.sigmoid(cd)))
             + jnp.sum(jnp.log(jax.nn.sigmoid(-nd))))
    out_ref[...] = jnp.full((1, 1), -total * (1.0 / B), jnp.float32)


_tc_kernel = pl.pallas_call(
    _tc_body,
    out_shape=jax.ShapeDtypeStruct((1, 1), jnp.float32),
)


@jax.jit
def kernel(target, context, noise_words, in_embed, out_embed):
    tbl = _tr_kernel(in_embed.T, out_embed.T)
    ctx_dots, noise_dots = _sc_kernel(
        target.astype(jnp.int32),
        context.astype(jnp.int32),
        noise_words.astype(jnp.int32).reshape(B * N_NEG),
        tbl,
    )
    loss = _tc_kernel(ctx_dots.reshape(B // 128, 128),
                      noise_dots.reshape(B * N_NEG // 128, 128))
    return loss[0, 0]


# speedup vs baseline: 3.0466x; 1.0138x over previous
import functools

import jax
import jax.numpy as jnp
from jax import lax
from jax.experimental import pallas as pl
from jax.experimental.pallas import tpu as pltpu
from jax.experimental.pallas import tpu_sc as plsc

VOCAB = 1000000
B = 16384
E = 64
N_NEG = 20
NC = 2
NS = 16
NW = NC * NS
BPW = B // NW
CB = 16
NCHUNK = BPW // CB
NIDX = CB * N_NEG

VCHUNK = 16384
NBLK = (VOCAB + VCHUNK - 1) // VCHUNK


def _tr_body(a_ref, b_ref, o_ref):
    o_ref[...] = jnp.concatenate([a_ref[...], b_ref[...]], axis=0).T


_tr_kernel = pl.pallas_call(
    _tr_body,
    grid=(NBLK,),
    in_specs=[
        pl.BlockSpec((E, VCHUNK), lambda i: (0, i)),
        pl.BlockSpec((E, VCHUNK), lambda i: (0, i)),
    ],
    out_specs=pl.BlockSpec((VCHUNK, 2 * E), lambda i: (i, 0)),
    out_shape=jax.ShapeDtypeStruct((NBLK * VCHUNK, 2 * E), jnp.float32),
)


def _sc_body(tgt_hbm, ctx_hbm, noise_hbm, tbl_hbm,
             ctxo_hbm, noiseo_hbm,
             tgt_idx, ctx_idx, noise_idx, t_buf, c_buf, n_buf,
             ctx_dots, noise_dots, sem0, sem1):
    wid = lax.axis_index("s") * NC + lax.axis_index("c")
    iota = lax.iota(jnp.int32, 16)

    pltpu.sync_copy(tgt_hbm.at[pl.ds(wid * BPW, BPW)], tgt_idx)
    pltpu.sync_copy(ctx_hbm.at[pl.ds(wid * BPW, BPW)], ctx_idx)
    pltpu.sync_copy(noise_hbm.at[pl.ds(wid * BPW * N_NEG, BPW * N_NEG)],
                    noise_idx)

    def fire(c, b, sem):
        pltpu.async_copy(tbl_hbm.at[tgt_idx.at[pl.ds(c * CB, CB)]],
                         t_buf.at[b], sem)
        pltpu.async_copy(tbl_hbm.at[ctx_idx.at[pl.ds(c * CB, CB)]],
                         c_buf.at[b], sem)
        for off, ln in ((0, 128), (128, 128), (256, 64)):
            pltpu.async_copy(
                tbl_hbm.at[noise_idx.at[pl.ds(c * NIDX + off, ln)]],
                n_buf.at[b, pl.ds(off, ln)], sem)

    def drain(b, sem):
        pltpu.make_async_copy(tbl_hbm.at[pl.ds(0, CB)], t_buf.at[b],
                              sem).wait()
        pltpu.make_async_copy(tbl_hbm.at[pl.ds(0, CB)], c_buf.at[b],
                              sem).wait()
        for off, ln in ((0, 128), (128, 128), (256, 64)):
            pltpu.make_async_copy(tbl_hbm.at[pl.ds(0, ln)],
                                  n_buf.at[b, pl.ds(off, ln)], sem).wait()

    def compute(c, b):
        t_slot = t_buf.at[b]
        c_slot = c_buf.at[b]
        n_slot = n_buf.at[b]
        lane15 = iota == 15

        def jbody(j, carry):
            t = [t_slot[j, pl.ds(16 * k, 16)] for k in range(E // 16)]

            def dot_store(row_ref, r, dots_ref, pos):
                x = [row_ref[r, pl.ds(E + 16 * k, 16)]
                     for k in range(E // 16)]
                p = (t[0] * x[0] + t[1] * x[1]) + (t[2] * x[2] + t[3] * x[3])
                cum = plsc.cumsum(p)
                idx = jnp.full((16,), 0, jnp.int32) + pos
                plsc.store_scatter(dots_ref, [idx], cum, mask=lane15)

            dot_store(c_slot, j, ctx_dots, c * CB + j)
            for s in range(N_NEG):
                dot_store(n_slot, j * N_NEG + s, noise_dots,
                          s * BPW + c * CB + j)
            return carry

        lax.fori_loop(0, CB, jbody, 0)

    fire(0, 0, sem0)

    def pipe_body(j, carry):
        c = 2 * j
        fire(c + 1, 1, sem1)
        drain(0, sem0)
        compute(c, 0)

        @pl.when(j < NCHUNK // 2 - 1)
        def _():
            fire(c + 2, 0, sem0)

        drain(1, sem1)
        compute(c + 1, 1)
        return carry

    lax.fori_loop(0, NCHUNK // 2, pipe_body, 0)

    pltpu.sync_copy(ctx_dots, ctxo_hbm.at[wid])
    pltpu.sync_copy(noise_dots, noiseo_hbm.at[wid])


_sc_kernel = functools.partial(
    pl.kernel,
    out_type=(
        jax.ShapeDtypeStruct((NW, BPW), jnp.float32),
        jax.ShapeDtypeStruct((NW, N_NEG * BPW), jnp.float32),
    ),
    mesh=plsc.VectorSubcoreMesh(core_axis_name="c", subcore_axis_name="s",
                                num_cores=NC, num_subcores=NS),
    compiler_params=pltpu.CompilerParams(needs_layout_passes=False),
    scratch_types=[
        pltpu.VMEM((BPW,), jnp.int32),
        pltpu.VMEM((BPW,), jnp.int32),
        pltpu.VMEM((BPW * N_NEG,), jnp.int32),
        pltpu.VMEM((2, CB, 2 * E), jnp.float32),
        pltpu.VMEM((2, CB, 2 * E), jnp.float32),
        pltpu.VMEM((2, NIDX, 2 * E), jnp.float32),
        pltpu.VMEM((BPW,), jnp.float32),
        pltpu.VMEM((N_NEG * BPW,), jnp.float32),
        pltpu.SemaphoreType.DMA,
        pltpu.SemaphoreType.DMA,
    ],
)(_sc_body)


def _tc_body(ctx_ref, noise_ref, out_ref):
    cd = ctx_ref[...]
    nd = noise_ref[...]
    total = (jnp.sum(jnp.log(jax.nn.sigmoid(cd)))
             + jnp.sum(jnp.log(jax.nn.sigmoid(-nd))))
    out_ref[...] = jnp.full((1, 1), -total * (1.0 / B), jnp.float32)


_tc_kernel = pl.pallas_call(
    _tc_body,
    out_shape=jax.ShapeDtypeStruct((1, 1), jnp.float32),
)


@jax.jit
def kernel(target, context, noise_words, in_embed, out_embed):
    tbl = _tr_kernel(in_embed.T, out_embed.T)
    ctx_dots, noise_dots = _sc_kernel(
        target.astype(jnp.int32),
        context.astype(jnp.int32),
        noise_words.astype(jnp.int32).reshape(B * N_NEG),
        tbl,
    )
    loss = _tc_kernel(ctx_dots.reshape(B // 128, 128),
                      noise_dots.reshape(B * N_NEG // 128, 128))
    return loss[0, 0]
